# baseline (device time: 25027 ns/iter reference)
import jax
import jax.numpy as jnp
from jax import lax
from jax.experimental import pallas as pl
from jax.experimental.pallas import tpu as pltpu

K = 16
NEG_INF = float("-inf")
N_BLK = 2
S1_ROUNDS = 3


def _topk_cols(xv, k):
    cols = []
    for _ in range(k):
        m = jnp.max(xv, axis=1, keepdims=True)
        cols.append(m)
        xv = jnp.where(xv == m, NEG_INF, xv)
    return cols


def _block_topk(xv):
    rows, n = xv.shape
    x3 = xv.reshape(rows, n // 128, 128)
    cands = []
    for _ in range(S1_ROUNDS):
        li = jnp.max(x3, axis=1)
        cands.append(li)
        x3 = jnp.where(x3 == li[:, None, :], NEG_INF, x3)
    cols = _topk_cols(jnp.concatenate(cands, axis=1), K)
    return jnp.concatenate(cols, axis=1), jnp.concatenate(cols[::-1], axis=1)


def _bitonic_desc(u):
    rows = u.shape[0]
    for d in (8, 4, 2, 1):
        g = u.reshape(rows, 16 // (2 * d), 2, d)
        hi = jnp.maximum(g[:, :, 0, :], g[:, :, 1, :])
        lo = jnp.minimum(g[:, :, 0, :], g[:, :, 1, :])
        u = jnp.stack([hi, lo], axis=2).reshape(rows, 16)
    return u


def _merge16(a_desc, b_asc):
    return _bitonic_desc(jnp.maximum(a_desc, b_asc))


def kernel(x):
    rows, n = x.shape

    def body(x_ref, out_ref, loc_ref, asc_ref, rem_ref, s_sem, r_sem):
        my_x = lax.axis_index("x")
        my_y = lax.axis_index("y")
        nbr = (1 - my_x, my_y)

        barrier_sem = pltpu.get_barrier_semaphore()
        pl.semaphore_signal(
            barrier_sem, inc=1, device_id=nbr,
            device_id_type=pl.DeviceIdType.MESH,
        )
        pl.semaphore_wait(barrier_sem, 1)

        desc, asc = _block_topk(x_ref[:, :])
        loc_ref[:, :] = desc
        asc_ref[:, :] = asc

        rdma = pltpu.make_async_remote_copy(
            src_ref=asc_ref,
            dst_ref=rem_ref,
            send_sem=s_sem,
            recv_sem=r_sem,
            device_id=nbr,
            device_id_type=pl.DeviceIdType.MESH,
        )
        rdma.start()
        rdma.wait()

        out_ref[:, :] = _merge16(loc_ref[:, :], rem_ref[:, :])

    return pl.pallas_call(
        body,
        out_shape=jax.ShapeDtypeStruct((rows, K), jnp.float32),
        in_specs=[pl.BlockSpec(memory_space=pltpu.VMEM)],
        out_specs=pl.BlockSpec(memory_space=pltpu.VMEM),
        scratch_shapes=[
            pltpu.VMEM((rows, K), jnp.float32),
            pltpu.VMEM((rows, K), jnp.float32),
            pltpu.VMEM((rows, K), jnp.float32),
            pltpu.SemaphoreType.DMA,
            pltpu.SemaphoreType.DMA,
        ],
        compiler_params=pltpu.CompilerParams(collective_id=0),
    )(x)


# device time: 16413 ns/iter; 1.5248x vs baseline; 1.5248x over previous
import jax
import jax.numpy as jnp
from jax import lax
from jax.experimental import pallas as pl
from jax.experimental.pallas import tpu as pltpu

K = 16
NEG_INF = float("-inf")


def _topk_desc(xv, k):
    cols = []
    for _ in range(k):
        m = jnp.max(xv, axis=1, keepdims=True)
        cols.append(m)
        xv = jnp.where(xv == m, NEG_INF, xv)
    return jnp.concatenate(cols, axis=1)


def _local_topk(xv, k):
    rows, n = xv.shape
    x3 = xv.reshape(rows, n // 128, 128)
    cands = []
    for _ in range(2):
        li = jnp.max(x3, axis=1)
        cands.append(li)
        x3 = jnp.where(x3 == li[:, None, :], NEG_INF, x3)
    return _topk_desc(jnp.concatenate(cands, axis=1), k)


def kernel(x):
    rows, n = x.shape

    def body(x_ref, out_ref, cand_ref, send_sem, recv_sem):
        my_x = lax.axis_index("x")
        my_y = lax.axis_index("y")
        nbr = (1 - my_x, my_y)

        barrier_sem = pltpu.get_barrier_semaphore()
        pl.semaphore_signal(
            barrier_sem, inc=1, device_id=nbr,
            device_id_type=pl.DeviceIdType.MESH,
        )
        pl.semaphore_wait(barrier_sem, 1)

        cand_ref[0] = _local_topk(x_ref[:, :], K)

        rdma = pltpu.make_async_remote_copy(
            src_ref=cand_ref.at[0],
            dst_ref=cand_ref.at[1],
            send_sem=send_sem,
            recv_sem=recv_sem,
            device_id=nbr,
            device_id_type=pl.DeviceIdType.MESH,
        )
        rdma.start()
        rdma.wait()

        merged = jnp.concatenate([cand_ref[0], cand_ref[1]], axis=1)
        out_ref[:, :] = _topk_desc(merged, K)

    return pl.pallas_call(
        body,
        out_shape=jax.ShapeDtypeStruct((rows, K), jnp.float32),
        in_specs=[pl.BlockSpec(memory_space=pltpu.VMEM)],
        out_specs=pl.BlockSpec(memory_space=pltpu.VMEM),
        scratch_shapes=[
            pltpu.VMEM((2, rows, K), jnp.float32),
            pltpu.SemaphoreType.DMA,
            pltpu.SemaphoreType.DMA,
        ],
        compiler_params=pltpu.CompilerParams(collective_id=0),
    )(x)
